# Initial kernel scaffold; baseline (speedup 1.0000x reference)
import jax
import jax.numpy as jnp
from jax.experimental import pallas as pl


def kernel(x, weight):
    raise NotImplementedError("write your pallas kernel here")


# speedup vs baseline: 1.0771x; 1.0771x over previous
import functools

import jax
import jax.numpy as jnp
from jax.experimental import pallas as pl

_DIM = 2048
_NE = 64
_K = 8
_BT = 512


def _gate_block(x_ref, w_ref, vals_ref, idx_ref):
    x = x_ref[...]
    w = w_ref[...]
    logits = jax.lax.dot_general(
        x, w, (((1,), (1,)), ((), ())), preferred_element_type=jnp.float32
    )
    lanes = jax.lax.broadcasted_iota(jnp.int32, (_BT, _NE), 1)
    neg_inf = jnp.float32(-jnp.inf)
    vals = []
    idxs = []
    l = logits
    for _ in range(_K):
        m = jnp.max(l, axis=1, keepdims=True)
        is_m = l == m
        idx = jnp.min(jnp.where(is_m, lanes, _NE), axis=1, keepdims=True)
        l = jnp.where(lanes == idx, neg_inf, l)
        vals.append(m)
        idxs.append(idx)
    top = jnp.concatenate(vals, axis=1)
    top_idx = jnp.concatenate(idxs, axis=1)
    s = jax.nn.sigmoid(top)
    s = s / jnp.sum(s, axis=1, keepdims=True)
    vals_ref[...] = s
    idx_ref[...] = top_idx


@jax.jit
def kernel(x, weight):
    t = x.shape[0]
    grid = (t // _BT,)
    vals, idx = pl.pallas_call(
        _gate_block,
        grid=grid,
        in_specs=[
            pl.BlockSpec((_BT, _DIM), lambda i: (i, 0)),
            pl.BlockSpec((_NE, _DIM), lambda i: (0, 0)),
        ],
        out_specs=[
            pl.BlockSpec((_BT, _K), lambda i: (i, 0)),
            pl.BlockSpec((_BT, _K), lambda i: (i, 0)),
        ],
        out_shape=[
            jax.ShapeDtypeStruct((t, _K), jnp.float32),
            jax.ShapeDtypeStruct((t, _K), jnp.int32),
        ],
    )(x, weight)
    return vals, idx
